# baseline (speedup 1.0000x reference)
import jax
import jax.numpy as jnp
from jax import lax
from jax.experimental import pallas as pl
from jax.experimental.pallas import tpu as pltpu
from jax.experimental.pallas import tpu_sc as plsc

_MAX_DIST = 1024
_PROJ_DIM = 32
_SEQ_LEN = 2048
_TROWS = 2 * _MAX_DIST - 1

_NC = 2
_NS = 16
_NW = _NC * _NS
_RPT = _SEQ_LEN // _NW
_LO = 960
_WBUF_ROWS = _LO + _MAX_DIST + _TROWS
_GROUP = 8


def _body(table_hbm, out_hbm, win_v, sem):
    w = lax.axis_index("c") * _NS + lax.axis_index("s")
    base = (_MAX_DIST - 1) - (_RPT * w + _RPT - 1)
    trow = _LO - base

    pltpu.async_copy(table_hbm, win_v.at[pl.ds(trow, _TROWS), :], sem).wait()

    h = jnp.maximum(0, -base)
    t = jnp.maximum(0, base + _RPT)
    first0 = win_v[trow, pl.ds(0, 16)]
    first1 = win_v[trow, pl.ds(16, 16)]
    last0 = win_v[trow + _TROWS - 1, pl.ds(0, 16)]
    last1 = win_v[trow + _TROWS - 1, pl.ds(16, 16)]

    def fill_head(m, carry):
        win_v[_LO + m, pl.ds(0, 16)] = first0
        win_v[_LO + m, pl.ds(16, 16)] = first1
        return carry

    def fill_tail(m, carry):
        r = _LO + _SEQ_LEN + _RPT - 2 - m
        win_v[r, pl.ds(0, 16)] = last0
        win_v[r, pl.ds(16, 16)] = last1
        return carry

    lax.fori_loop(0, h, fill_head, 0)
    lax.fori_loop(0, t, fill_tail, 0)

    row0 = _RPT * w

    def out_group(g, carry):
        handles = []
        for r in range(_GROUP):
            l = g * _GROUP + r
            handles.append(
                pltpu.async_copy(
                    win_v.at[pl.ds(_LO + _RPT - 1 - l, _SEQ_LEN), :],
                    out_hbm.at[row0 + l],
                    sem,
                )
            )
        for hd in handles:
            hd.wait()
        return carry

    lax.fori_loop(0, _RPT // _GROUP, out_group, 0)


def kernel(seq_len, table):
    del seq_len
    run = pl.kernel(
        _body,
        mesh=plsc.VectorSubcoreMesh(core_axis_name="c", subcore_axis_name="s"),
        out_type=jax.ShapeDtypeStruct((_SEQ_LEN, _SEQ_LEN, _PROJ_DIM), jnp.float32),
        scratch_types=[
            pltpu.VMEM((_WBUF_ROWS, _PROJ_DIM), jnp.float32),
            pltpu.SemaphoreType.DMA,
        ],
        compiler_params=pltpu.CompilerParams(use_tc_tiling_on_sc=False),
    )
    return run(table)


# speedup vs baseline: 1.7699x; 1.7699x over previous
import jax
import jax.numpy as jnp
from jax import lax
from jax.experimental import pallas as pl
from jax.experimental.pallas import tpu as pltpu
from jax.experimental.pallas import tpu_sc as plsc

_MAX_DIST = 1024
_PROJ_DIM = 32
_SEQ_LEN = 2048

_NC = 2
_NS = 16
_NW = _NC * _NS
_RPT = _SEQ_LEN // _NW
_WCOLS = _SEQ_LEN + _RPT


def _body(tbl_hbm, out_hbm, win_v, st0, st1, sem):
    w = lax.axis_index("c") * _NS + lax.axis_index("s")
    base = (_MAX_DIST - 1) - (_RPT * w + _RPT - 1)

    pltpu.async_copy(
        tbl_hbm.at[:, pl.ds(_MAX_DIST + base, _WCOLS)], win_v, sem
    ).wait()

    row0 = _RPT * w

    def shuffle_quarter(buf, ph, s_l):
        def shuf_p(p_lo, cc):
            p = ph * 8 + p_lo

            def shuf_jh(jh, cc2):
                qrow = jh * 8 + p_lo
                col0 = s_l + jh * 128
                for c8 in range(8):
                    buf[qrow, pl.ds(c8 * 16, 16)] = win_v[
                        p, pl.ds(col0 + c8 * 16, 16)
                    ]
                return cc2

            lax.fori_loop(0, 16, shuf_jh, 0)
            return cc

        lax.fori_loop(0, 8, shuf_p, 0)

    def per_row(l, carry):
        s_l = _RPT - 1 - l
        i = row0 + l
        shuffle_quarter(st0, 0, s_l)
        h0 = pltpu.async_copy(st0, out_hbm.at[i, pl.ds(0, 128), :], sem)
        shuffle_quarter(st1, 1, s_l)
        h1 = pltpu.async_copy(st1, out_hbm.at[i, pl.ds(128, 128), :], sem)
        h0.wait()
        shuffle_quarter(st0, 2, s_l)
        h2 = pltpu.async_copy(st0, out_hbm.at[i, pl.ds(256, 128), :], sem)
        h1.wait()
        shuffle_quarter(st1, 3, s_l)
        h3 = pltpu.async_copy(st1, out_hbm.at[i, pl.ds(384, 128), :], sem)
        h2.wait()
        h3.wait()
        return carry

    lax.fori_loop(0, _RPT, per_row, 0)


def kernel(seq_len, table):
    del seq_len
    tbl_t = jnp.pad(table.T, ((0, 0), (_MAX_DIST, _MAX_DIST + 1)), mode="edge")
    run = pl.kernel(
        _body,
        mesh=plsc.VectorSubcoreMesh(core_axis_name="c", subcore_axis_name="s"),
        out_type=jax.ShapeDtypeStruct((_SEQ_LEN, 512, 128), jnp.float32),
        scratch_types=[
            pltpu.VMEM((_PROJ_DIM, _WCOLS), jnp.float32),
            pltpu.VMEM((128, 128), jnp.float32),
            pltpu.VMEM((128, 128), jnp.float32),
            pltpu.SemaphoreType.DMA,
        ],
        compiler_params=pltpu.CompilerParams(use_tc_tiling_on_sc=False),
    )
    r = run(tbl_t)
    r5 = r.reshape(_SEQ_LEN, 4, 16, 8, 128)
    r5 = r5.transpose(0, 2, 4, 1, 3)
    return r5.reshape(_SEQ_LEN, _SEQ_LEN, _PROJ_DIM)


# speedup vs baseline: 1.8144x; 1.0251x over previous
import jax
import jax.numpy as jnp
from jax import lax
from jax.experimental import pallas as pl
from jax.experimental.pallas import tpu as pltpu
from jax.experimental.pallas import tpu_sc as plsc

_MAX_DIST = 1024
_PROJ_DIM = 32
_SEQ_LEN = 2048

_NC = 2
_NS = 16
_NW = _NC * _NS
_RPT = _SEQ_LEN // _NW
_WCOLS = _SEQ_LEN + _RPT


def _body(tbl_hbm, out_hbm, win_v, st0, st1, sem):
    w = lax.axis_index("c") * _NS + lax.axis_index("s")
    base = (_MAX_DIST - 1) - (_RPT * w + _RPT - 1)

    pltpu.async_copy(
        tbl_hbm.at[:, pl.ds(_MAX_DIST + base, _WCOLS)], win_v, sem
    ).wait()

    row0 = _RPT * w

    def shuffle_quarter(buf, ph, s_l):
        def shuf_jh(jh, cc):
            qrow0 = jh * 8
            col0 = s_l + jh * 128
            for p_lo in range(8):
                p = ph * 8 + p_lo
                for c8 in range(8):
                    buf[qrow0 + p_lo, pl.ds(c8 * 16, 16)] = win_v[
                        p, pl.ds(col0 + c8 * 16, 16)
                    ]
            return cc

        lax.fori_loop(0, 16, shuf_jh, 0)

    def per_row(l, carry):
        s_l = _RPT - 1 - l
        i = row0 + l
        shuffle_quarter(st0, 0, s_l)
        h0 = pltpu.async_copy(st0, out_hbm.at[i, pl.ds(0, 128), :], sem)
        shuffle_quarter(st1, 1, s_l)
        h1 = pltpu.async_copy(st1, out_hbm.at[i, pl.ds(128, 128), :], sem)
        h0.wait()
        shuffle_quarter(st0, 2, s_l)
        h2 = pltpu.async_copy(st0, out_hbm.at[i, pl.ds(256, 128), :], sem)
        h1.wait()
        shuffle_quarter(st1, 3, s_l)
        h3 = pltpu.async_copy(st1, out_hbm.at[i, pl.ds(384, 128), :], sem)
        h2.wait()
        h3.wait()
        return carry

    lax.fori_loop(0, _RPT, per_row, 0)


def kernel(seq_len, table):
    del seq_len
    tbl_t = jnp.pad(table.T, ((0, 0), (_MAX_DIST, _MAX_DIST + 1)), mode="edge")
    run = pl.kernel(
        _body,
        mesh=plsc.VectorSubcoreMesh(core_axis_name="c", subcore_axis_name="s"),
        out_type=jax.ShapeDtypeStruct((_SEQ_LEN, 512, 128), jnp.float32),
        scratch_types=[
            pltpu.VMEM((_PROJ_DIM, _WCOLS), jnp.float32),
            pltpu.VMEM((128, 128), jnp.float32),
            pltpu.VMEM((128, 128), jnp.float32),
            pltpu.SemaphoreType.DMA,
        ],
        compiler_params=pltpu.CompilerParams(use_tc_tiling_on_sc=False),
    )
    r = run(tbl_t)
    r5 = r.reshape(_SEQ_LEN, 4, 16, 8, 128)
    r5 = r5.transpose(0, 2, 4, 1, 3)
    return r5.reshape(_SEQ_LEN, _SEQ_LEN, _PROJ_DIM)


# speedup vs baseline: 5.7074x; 3.1456x over previous
import jax
import jax.numpy as jnp
from jax import lax
from jax.experimental import pallas as pl
from jax.experimental.pallas import tpu as pltpu
from jax.experimental.pallas import tpu_sc as plsc

_MAX_DIST = 1024
_PROJ_DIM = 32
_SEQ_LEN = 2048

_NC = 2
_NS = 16
_NW = _NC * _NS
_RPT = _SEQ_LEN // _NW
_WCOLS = _SEQ_LEN + _RPT


def _body(tbl_hbm, out_hbm, win_v, st0, st1, sem):
    w = lax.axis_index("c") * _NS + lax.axis_index("s")
    base = (_MAX_DIST - 1) - (_RPT * w + _RPT - 1)

    pltpu.async_copy(
        tbl_hbm.at[:, pl.ds(_MAX_DIST + base, _WCOLS)], win_v, sem
    ).wait()

    row0 = _RPT * w

    def shuffle_quarter(buf, ph, s_l):
        def shuf_jh(jh, cc):
            qrow0 = jh * 8
            col0 = s_l + jh * 128
            for p_lo in range(8):
                p = ph * 8 + p_lo
                vs = [win_v[p, pl.ds(col0 + c8 * 16, 16)] for c8 in range(8)]
                for c8 in range(8):
                    buf[qrow0 + p_lo, pl.ds(c8 * 16, 16)] = vs[c8]
            return cc

        lax.fori_loop(0, 16, shuf_jh, 0)

    def per_row(l, carry):
        s_l = _RPT - 1 - l
        i = row0 + l
        shuffle_quarter(st0, 0, s_l)
        h0 = pltpu.async_copy(st0, out_hbm.at[i, pl.ds(0, 128), :], sem)
        shuffle_quarter(st1, 1, s_l)
        h1 = pltpu.async_copy(st1, out_hbm.at[i, pl.ds(128, 128), :], sem)
        h0.wait()
        shuffle_quarter(st0, 2, s_l)
        h2 = pltpu.async_copy(st0, out_hbm.at[i, pl.ds(256, 128), :], sem)
        h1.wait()
        shuffle_quarter(st1, 3, s_l)
        h3 = pltpu.async_copy(st1, out_hbm.at[i, pl.ds(384, 128), :], sem)
        h2.wait()
        h3.wait()
        return carry

    lax.fori_loop(0, _RPT, per_row, 0)


def kernel(seq_len, table):
    del seq_len
    tbl_t = jnp.pad(table.T, ((0, 0), (_MAX_DIST, _MAX_DIST + 1)), mode="edge")
    run = pl.kernel(
        _body,
        mesh=plsc.VectorSubcoreMesh(core_axis_name="c", subcore_axis_name="s"),
        out_type=jax.ShapeDtypeStruct((_SEQ_LEN, 512, 128), jnp.float32),
        scratch_types=[
            pltpu.VMEM((_PROJ_DIM, _WCOLS), jnp.float32),
            pltpu.VMEM((128, 128), jnp.float32),
            pltpu.VMEM((128, 128), jnp.float32),
            pltpu.SemaphoreType.DMA,
        ],
        compiler_params=pltpu.CompilerParams(use_tc_tiling_on_sc=False),
    )
    r = run(tbl_t)
    r5 = r.reshape(_SEQ_LEN, 4, 16, 8, 128)
    r5 = r5.transpose(0, 2, 4, 1, 3)
    return r5.reshape(_SEQ_LEN, _SEQ_LEN, _PROJ_DIM)


# speedup vs baseline: 6.1382x; 1.0755x over previous
import jax
import jax.numpy as jnp
from jax import lax
from jax.experimental import pallas as pl
from jax.experimental.pallas import tpu as pltpu
from jax.experimental.pallas import tpu_sc as plsc

_MAX_DIST = 1024
_PROJ_DIM = 32
_SEQ_LEN = 2048

_NC = 2
_NS = 16
_NW = _NC * _NS
_RPT = _SEQ_LEN // _NW
_WCOLS = _SEQ_LEN + _RPT


def _body(tbl_hbm, out_hbm, win_v, st0, st1, sem):
    w = lax.axis_index("c") * _NS + lax.axis_index("s")
    base = (_MAX_DIST - 1) - (_RPT * w + _RPT - 1)

    pltpu.async_copy(
        tbl_hbm.at[:, pl.ds(_MAX_DIST + base, _WCOLS)], win_v, sem
    ).wait()

    row0 = _RPT * w

    def shuffle_quarter(buf, ph, s_l):
        def shuf_jh(jh, cc):
            qrow0 = jh * 8
            col0 = s_l + jh * 128
            for p_lo in range(8):
                p = ph * 8 + p_lo
                vs = [win_v[p, pl.ds(col0 + c8 * 16, 16)] for c8 in range(8)]
                for c8 in range(8):
                    buf[qrow0 + p_lo, pl.ds(c8 * 16, 16)] = vs[c8]
            return cc

        lax.fori_loop(0, 16, shuf_jh, 0)

    def per_rows(l4, carry):
        pending = []
        for r4 in range(4):
            l = l4 * 4 + r4
            s_l = _RPT - 1 - l
            i = row0 + l
            for ph in range(4):
                buf = st0 if ph % 2 == 0 else st1
                if len(pending) >= 2:
                    pending.pop(0).wait()
                shuffle_quarter(buf, ph, s_l)
                pending.append(
                    pltpu.async_copy(
                        buf, out_hbm.at[i, pl.ds(ph * 128, 128), :], sem
                    )
                )
        for hd in pending:
            hd.wait()
        return carry

    lax.fori_loop(0, _RPT // 4, per_rows, 0)


def kernel(seq_len, table):
    del seq_len
    tbl_t = jnp.pad(table.T, ((0, 0), (_MAX_DIST, _MAX_DIST + 1)), mode="edge")
    run = pl.kernel(
        _body,
        mesh=plsc.VectorSubcoreMesh(core_axis_name="c", subcore_axis_name="s"),
        out_type=jax.ShapeDtypeStruct((_SEQ_LEN, 512, 128), jnp.float32),
        scratch_types=[
            pltpu.VMEM((_PROJ_DIM, _WCOLS), jnp.float32),
            pltpu.VMEM((128, 128), jnp.float32),
            pltpu.VMEM((128, 128), jnp.float32),
            pltpu.SemaphoreType.DMA,
        ],
        compiler_params=pltpu.CompilerParams(use_tc_tiling_on_sc=False),
    )
    r = run(tbl_t)
    r5 = r.reshape(_SEQ_LEN, 4, 16, 8, 128)
    r5 = r5.transpose(0, 2, 4, 1, 3)
    return r5.reshape(_SEQ_LEN, _SEQ_LEN, _PROJ_DIM)


# speedup vs baseline: 6.1514x; 1.0022x over previous
import jax
import jax.numpy as jnp
from jax import lax
from jax.experimental import pallas as pl
from jax.experimental.pallas import tpu as pltpu
from jax.experimental.pallas import tpu_sc as plsc

_MAX_DIST = 1024
_PROJ_DIM = 32
_SEQ_LEN = 2048

_NC = 2
_NS = 16
_NW = _NC * _NS
_RPT = _SEQ_LEN // _NW
_WCOLS = _SEQ_LEN + _RPT


def _body(tbl_hbm, out_hbm, win_v, st0, st1, sem):
    w = lax.axis_index("c") * _NS + lax.axis_index("s")
    base = (_MAX_DIST - 1) - (_RPT * w + _RPT - 1)

    pltpu.async_copy(
        tbl_hbm.at[:, pl.ds(_MAX_DIST + base, _WCOLS)], win_v, sem
    ).wait()

    row0 = _RPT * w

    def shuffle_quarter(buf, ph, s_l):
        def shuf_jh(jh, cc):
            qrow0 = jh * 8
            col0 = s_l + jh * 128
            for p_lo in range(8):
                p = ph * 8 + p_lo
                vs = [win_v[p, pl.ds(col0 + c8 * 16, 16)] for c8 in range(8)]
                for c8 in range(8):
                    buf[qrow0 + p_lo, pl.ds(c8 * 16, 16)] = vs[c8]
            return cc

        lax.fori_loop(0, 16, shuf_jh, 0)

    def per_rows(l4, carry):
        pending = []
        for r4 in range(8):
            l = l4 * 8 + r4
            s_l = _RPT - 1 - l
            i = row0 + l
            for ph in range(4):
                buf = st0 if ph % 2 == 0 else st1
                if len(pending) >= 2:
                    pending.pop(0).wait()
                shuffle_quarter(buf, ph, s_l)
                pending.append(
                    pltpu.async_copy(
                        buf, out_hbm.at[i, pl.ds(ph * 128, 128), :], sem
                    )
                )
        for hd in pending:
            hd.wait()
        return carry

    lax.fori_loop(0, _RPT // 8, per_rows, 0)


def kernel(seq_len, table):
    del seq_len
    tbl_t = jnp.pad(table.T, ((0, 0), (_MAX_DIST, _MAX_DIST + 1)), mode="edge")
    run = pl.kernel(
        _body,
        mesh=plsc.VectorSubcoreMesh(core_axis_name="c", subcore_axis_name="s"),
        out_type=jax.ShapeDtypeStruct((_SEQ_LEN, 512, 128), jnp.float32),
        scratch_types=[
            pltpu.VMEM((_PROJ_DIM, _WCOLS), jnp.float32),
            pltpu.VMEM((128, 128), jnp.float32),
            pltpu.VMEM((128, 128), jnp.float32),
            pltpu.SemaphoreType.DMA,
        ],
        compiler_params=pltpu.CompilerParams(use_tc_tiling_on_sc=False),
    )
    r = run(tbl_t)
    r5 = r.reshape(_SEQ_LEN, 4, 16, 8, 128)
    r5 = r5.transpose(0, 2, 4, 1, 3)
    return r5.reshape(_SEQ_LEN, _SEQ_LEN, _PROJ_DIM)
